# baseline (speedup 1.0000x reference)
import functools

import jax
import jax.numpy as jnp
from jax import lax
from jax.experimental import pallas as pl
from jax.experimental.pallas import tpu as pltpu
from jax.experimental.pallas import tpu_sc as plsc

N_NODES = 10000
N_EDGES = 320000
D_NODE = 128
D_EDGE = 16
D_OUT = 128
DH = D_NODE // 2

NC = 2
NS = 16
CHUNK = 128
N_CHUNKS = N_EDGES // CHUNK
CHUNKS_PER_TILE = N_CHUNKS // NS
BLK = 12
N_BLKS = CHUNKS_PER_TILE // BLK
TAIL_CHUNKS = N_CHUNKS - NS * CHUNKS_PER_TILE
NBUF = 8
ROWS_PER_TILE = (N_NODES // NS) // 8 * 8
ROW_TAIL = N_NODES - NS * ROWS_PER_TILE

_MESH = dict(core_axis_name="c", subcore_axis_name="s")


def _sc_node_accumulate(nf0, nf1, src2d, dst2d, zeros_d):
    @functools.partial(
        pl.kernel,
        out_type=jax.ShapeDtypeStruct((NC, N_NODES, DH), jnp.float32),
        mesh=plsc.VectorSubcoreMesh(**_MESH),
        scratch_types=[
            pltpu.VMEM_SHARED((N_NODES, DH), jnp.float32),
            pltpu.VMEM((2, BLK, CHUNK), jnp.int32),
            pltpu.VMEM((2, BLK, CHUNK), jnp.int32),
            pltpu.VMEM((1, CHUNK), jnp.int32),
            pltpu.VMEM((1, CHUNK), jnp.int32),
            pltpu.VMEM((CHUNK, DH), jnp.float32),
            pltpu.VMEM((CHUNK, DH), jnp.float32),
            pltpu.VMEM((CHUNK, DH), jnp.float32),
            pltpu.VMEM((CHUNK, DH), jnp.float32),
            pltpu.VMEM((CHUNK, DH), jnp.float32),
            pltpu.VMEM((CHUNK, DH), jnp.float32),
            pltpu.VMEM((CHUNK, DH), jnp.float32),
            pltpu.VMEM((CHUNK, DH), jnp.float32),
            pltpu.SemaphoreType.DMA,
            pltpu.SemaphoreType.DMA,
            pltpu.SemaphoreType.DMA,
        ],
        compiler_params=pltpu.CompilerParams(use_tc_tiling_on_sc=False),
    )
    def k(nf0_hbm, nf1_hbm, src_hbm, dst_hbm, zd_hbm, acc_out,
          acc_sh, idxs, idxd, idxts, idxtd,
          rowb0, rowb1, rowb2, rowb3, rowb4, rowb5, rowb6, rowb7,
          gsem, ssem, isem):
        rowb = [rowb0, rowb1, rowb2, rowb3, rowb4, rowb5, rowb6, rowb7]
        c = lax.axis_index("c")
        s = lax.axis_index("s")

        r0 = s * ROWS_PER_TILE
        rt0 = NS * ROWS_PER_TILE
        pltpu.sync_copy(zd_hbm, acc_sh.at[pl.ds(r0, ROWS_PER_TILE)])

        @pl.when(s == 0)
        def _zero_tail():
            pltpu.sync_copy(zd_hbm.at[pl.ds(0, ROW_TAIL)],
                            acc_sh.at[pl.ds(rt0, ROW_TAIL)])

        plsc.subcore_barrier()

        chunk0 = s * CHUNKS_PER_TILE

        pltpu.async_copy(src_hbm.at[pl.ds(chunk0, BLK)], idxs.at[0], isem)
        pltpu.async_copy(dst_hbm.at[pl.ds(chunk0, BLK)], idxd.at[0], isem)

        def block_body(b, carry):
            row0 = chunk0 + b * BLK
            p = lax.rem(b, 2)
            pltpu.make_async_copy(src_hbm.at[pl.ds(row0, BLK)],
                                  idxs.at[p], isem).wait()
            pltpu.make_async_copy(dst_hbm.at[pl.ds(row0, BLK)],
                                  idxd.at[p], isem).wait()

            @pl.when(b + 1 < N_BLKS)
            def _prefetch_idx():
                pltpu.async_copy(src_hbm.at[pl.ds(row0 + BLK, BLK)],
                                 idxs.at[1 - p], isem)
                pltpu.async_copy(dst_hbm.at[pl.ds(row0 + BLK, BLK)],
                                 idxd.at[1 - p], isem)

            sd = {}

            def start_g(i):
                @pl.when(c == 0)
                def _g0(i=i):
                    pltpu.async_copy(
                        nf0_hbm.at[idxs.at[p, i]], rowb[i % NBUF], gsem)

                @pl.when(c == 1)
                def _g1(i=i):
                    pltpu.async_copy(
                        nf1_hbm.at[idxs.at[p, i]], rowb[i % NBUF], gsem)

            def wait_g(i):
                pltpu.make_async_copy(
                    nf0_hbm.at[idxs.at[p, i]], rowb[i % NBUF], gsem).wait()

            for i in range(NBUF):
                start_g(i)

            for i in range(BLK):
                q = i % NBUF
                wait_g(i)
                sd[i] = pltpu.async_copy(
                    rowb[q], acc_sh.at[idxd.at[p, i]], ssem, add=True)
                if i + NBUF < BLK:
                    sd[i].wait()
                    start_g(i + NBUF)

            for i in range(BLK - NBUF, BLK):
                sd[i].wait()

            return carry

        lax.fori_loop(0, N_BLKS, block_body, 0)

        @pl.when(s < TAIL_CHUNKS)
        def _tail():
            row = NS * CHUNKS_PER_TILE + s
            pltpu.sync_copy(src_hbm.at[pl.ds(row, 1)], idxts)
            pltpu.sync_copy(dst_hbm.at[pl.ds(row, 1)], idxtd)

            @pl.when(c == 0)
            def _tg0():
                pltpu.async_copy(nf0_hbm.at[idxts.at[0]], rowb0, gsem)

            @pl.when(c == 1)
            def _tg1():
                pltpu.async_copy(nf1_hbm.at[idxts.at[0]], rowb0, gsem)

            pltpu.make_async_copy(
                nf0_hbm.at[idxts.at[0]], rowb0, gsem).wait()
            pltpu.sync_copy(rowb0, acc_sh.at[idxtd.at[0]], add=True)

        plsc.subcore_barrier()

        pltpu.sync_copy(acc_sh.at[pl.ds(r0, ROWS_PER_TILE)],
                        acc_out.at[c, pl.ds(r0, ROWS_PER_TILE)])

        @pl.when(s == 0)
        def _write_tail():
            pltpu.sync_copy(acc_sh.at[pl.ds(rt0, ROW_TAIL)],
                            acc_out.at[c, pl.ds(rt0, ROW_TAIL)])

    return k(nf0, nf1, src2d, dst2d, zeros_d)


def _sc_edge_accumulate(ef, dst2d, zeros_e, ones_e):
    @functools.partial(
        pl.kernel,
        out_type=[
            jax.ShapeDtypeStruct((NC, N_NODES, D_EDGE), jnp.float32),
            jax.ShapeDtypeStruct((NC, N_NODES, D_EDGE), jnp.float32),
        ],
        mesh=plsc.VectorSubcoreMesh(**_MESH),
        scratch_types=[
            pltpu.VMEM_SHARED((N_NODES, D_EDGE), jnp.float32),
            pltpu.VMEM_SHARED((N_NODES, D_EDGE), jnp.float32),
            pltpu.VMEM((2, BLK, CHUNK), jnp.int32),
            pltpu.VMEM((1, CHUNK), jnp.int32),
            pltpu.VMEM((2, BLK // 2, CHUNK, D_EDGE), jnp.float32),
            pltpu.VMEM((CHUNK, D_EDGE), jnp.float32),
            pltpu.SemaphoreType.DMA,
            pltpu.SemaphoreType.DMA,
            pltpu.SemaphoreType.DMA,
            pltpu.SemaphoreType.DMA,
        ],
        compiler_params=pltpu.CompilerParams(use_tc_tiling_on_sc=False),
    )
    def k(ef_hbm, dst_hbm, ze_hbm, oe_hbm, efacc_out, one_out,
          efacc_sh, one_sh, dstblk, idxtd, efblk, onesbuf,
          efsem, esem, osem, dsem):
        c = lax.axis_index("c")
        s = lax.axis_index("s")

        r0 = s * ROWS_PER_TILE
        rt0 = NS * ROWS_PER_TILE
        pltpu.sync_copy(ze_hbm, efacc_sh.at[pl.ds(r0, ROWS_PER_TILE)])
        pltpu.sync_copy(ze_hbm, one_sh.at[pl.ds(r0, ROWS_PER_TILE)])

        @pl.when(s == 0)
        def _zero_tail():
            pltpu.sync_copy(ze_hbm.at[pl.ds(0, ROW_TAIL)],
                            efacc_sh.at[pl.ds(rt0, ROW_TAIL)])
            pltpu.sync_copy(ze_hbm.at[pl.ds(0, ROW_TAIL)],
                            one_sh.at[pl.ds(rt0, ROW_TAIL)])

        pltpu.sync_copy(oe_hbm, onesbuf)
        plsc.subcore_barrier()

        chunk0 = s * CHUNKS_PER_TILE

        pltpu.async_copy(dst_hbm.at[pl.ds(chunk0, BLK)], dstblk.at[0], dsem)
        for t in range(BLK // 2):
            pltpu.async_copy(
                ef_hbm.at[pl.ds((chunk0 + 2 * t + c) * CHUNK, CHUNK)],
                efblk.at[0, t], efsem)

        def block_body(b, carry):
            row0 = chunk0 + b * BLK
            p = lax.rem(b, 2)

            @pl.when(b + 1 < N_BLKS)
            def _prefetch():
                pltpu.async_copy(dst_hbm.at[pl.ds(row0 + BLK, BLK)],
                                 dstblk.at[1 - p], dsem)
                for t in range(BLK // 2):
                    pltpu.async_copy(
                        ef_hbm.at[pl.ds((row0 + BLK + 2 * t + c) * CHUNK,
                                        CHUNK)],
                        efblk.at[1 - p, t], efsem)

            pltpu.make_async_copy(dst_hbm.at[pl.ds(row0, BLK)],
                                  dstblk.at[p], dsem).wait()

            e2, o2 = {}, {}
            for t in range(BLK // 2):
                pltpu.make_async_copy(
                    ef_hbm.at[pl.ds(row0 * CHUNK, CHUNK)],
                    efblk.at[p, t], efsem).wait()
                e2[t] = pltpu.async_copy(
                    efblk.at[p, t], efacc_sh.at[dstblk.at[p, 2 * t + c]],
                    esem, add=True)
                o2[t] = pltpu.async_copy(
                    onesbuf, one_sh.at[dstblk.at[p, 2 * t + c]], osem,
                    add=True)
            for t in range(BLK // 2):
                e2[t].wait()
                o2[t].wait()
            return carry

        lax.fori_loop(0, N_BLKS, block_body, 0)

        @pl.when(jnp.logical_and(s < TAIL_CHUNKS, c == 0))
        def _tail():
            row = NS * CHUNKS_PER_TILE + s
            off = row * CHUNK
            pltpu.sync_copy(dst_hbm.at[pl.ds(row, 1)], idxtd)
            pltpu.sync_copy(ef_hbm.at[pl.ds(off, CHUNK)], efblk.at[0, 0])
            pltpu.sync_copy(efblk.at[0, 0], efacc_sh.at[idxtd.at[0]], add=True)
            pltpu.sync_copy(onesbuf, one_sh.at[idxtd.at[0]], add=True)

        plsc.subcore_barrier()

        pltpu.sync_copy(efacc_sh.at[pl.ds(r0, ROWS_PER_TILE)],
                        efacc_out.at[c, pl.ds(r0, ROWS_PER_TILE)])
        pltpu.sync_copy(one_sh.at[pl.ds(r0, ROWS_PER_TILE)],
                        one_out.at[c, pl.ds(r0, ROWS_PER_TILE)])

        @pl.when(s == 0)
        def _write_tail():
            pltpu.sync_copy(efacc_sh.at[pl.ds(rt0, ROW_TAIL)],
                            efacc_out.at[c, pl.ds(rt0, ROW_TAIL)])
            pltpu.sync_copy(one_sh.at[pl.ds(rt0, ROW_TAIL)],
                            one_out.at[c, pl.ds(rt0, ROW_TAIL)])

    return k(ef, dst2d, zeros_e, ones_e)


def _tc_combine(acc, efacc, one, W_node, W_edge, b_node, b_edge):
    BR = 1000
    grid = (N_NODES // BR,)

    def body(acc_ref, ef_ref, one_ref, wn_ref, we_ref, bn_ref, be_ref, out_ref):
        wn = wn_ref[...]
        cnt = one_ref[0][:, 0:1] + one_ref[1][:, 0:1]
        out_ref[...] = (
            jnp.dot(acc_ref[0], wn[:DH], preferred_element_type=jnp.float32)
            + jnp.dot(acc_ref[1], wn[DH:], preferred_element_type=jnp.float32)
            + jnp.dot(ef_ref[0] + ef_ref[1], we_ref[...],
                      preferred_element_type=jnp.float32)
            + cnt * (bn_ref[...] + be_ref[...])
        )

    return pl.pallas_call(
        body,
        grid=grid,
        in_specs=[
            pl.BlockSpec((NC, BR, DH), lambda i: (0, i, 0)),
            pl.BlockSpec((NC, BR, D_EDGE), lambda i: (0, i, 0)),
            pl.BlockSpec((NC, BR, D_EDGE), lambda i: (0, i, 0)),
            pl.BlockSpec((D_NODE, D_OUT), lambda i: (0, 0)),
            pl.BlockSpec((D_EDGE, D_OUT), lambda i: (0, 0)),
            pl.BlockSpec((1, D_OUT), lambda i: (0, 0)),
            pl.BlockSpec((1, D_OUT), lambda i: (0, 0)),
        ],
        out_specs=pl.BlockSpec((BR, D_OUT), lambda i: (i, 0)),
        out_shape=jax.ShapeDtypeStruct((N_NODES, D_OUT), jnp.float32),
    )(acc, efacc, one, W_node, W_edge,
      b_node.reshape(1, D_OUT), b_edge.reshape(1, D_OUT))


def kernel(node_feats, edge_index, edge_feats, W_node, b_node, W_edge, b_edge):
    ei = edge_index.astype(jnp.int32)
    src2d = ei[0].reshape(N_CHUNKS, CHUNK)
    dst2d = ei[1].reshape(N_CHUNKS, CHUNK)
    nf0 = node_feats[:, :DH]
    nf1 = node_feats[:, DH:]
    zeros_d = jnp.zeros((ROWS_PER_TILE, DH), jnp.float32)
    zeros_e = jnp.zeros((ROWS_PER_TILE, D_EDGE), jnp.float32)
    ones_e = jnp.ones((CHUNK, D_EDGE), jnp.float32)
    acc = _sc_node_accumulate(nf0, nf1, src2d, dst2d, zeros_d)
    efacc, one = _sc_edge_accumulate(edge_feats, dst2d, zeros_e, ones_e)
    return _tc_combine(acc, efacc, one, W_node, W_edge, b_node, b_edge)


# speedup vs baseline: 1.0043x; 1.0043x over previous
import functools

import jax
import jax.numpy as jnp
from jax import lax
from jax.experimental import pallas as pl
from jax.experimental.pallas import tpu as pltpu
from jax.experimental.pallas import tpu_sc as plsc

N_NODES = 10000
N_EDGES = 320000
D_NODE = 128
D_EDGE = 16
D_OUT = 128
DH = D_NODE // 2

NC = 2
NS = 16
CHUNK = 128
N_CHUNKS = N_EDGES // CHUNK
CHUNKS_PER_TILE = N_CHUNKS // NS
BLK = 12
N_BLKS = CHUNKS_PER_TILE // BLK
TAIL_CHUNKS = N_CHUNKS - NS * CHUNKS_PER_TILE
NBUF = 6
ROWS_PER_TILE = (N_NODES // NS) // 8 * 8
ROW_TAIL = N_NODES - NS * ROWS_PER_TILE

_MESH = dict(core_axis_name="c", subcore_axis_name="s")


def _sc_node_accumulate(nf0, nf1, src2d, dst2d, zeros_d):
    @functools.partial(
        pl.kernel,
        out_type=jax.ShapeDtypeStruct((NC, N_NODES, DH), jnp.float32),
        mesh=plsc.VectorSubcoreMesh(**_MESH),
        scratch_types=[
            pltpu.VMEM_SHARED((N_NODES, DH), jnp.float32),
            pltpu.VMEM((2, BLK, CHUNK), jnp.int32),
            pltpu.VMEM((2, BLK, CHUNK), jnp.int32),
            pltpu.VMEM((1, CHUNK), jnp.int32),
            pltpu.VMEM((1, CHUNK), jnp.int32),
            pltpu.VMEM((CHUNK, DH), jnp.float32),
            pltpu.VMEM((CHUNK, DH), jnp.float32),
            pltpu.VMEM((CHUNK, DH), jnp.float32),
            pltpu.VMEM((CHUNK, DH), jnp.float32),
            pltpu.VMEM((CHUNK, DH), jnp.float32),
            pltpu.VMEM((CHUNK, DH), jnp.float32),
            pltpu.SemaphoreType.DMA,
            pltpu.SemaphoreType.DMA,
            pltpu.SemaphoreType.DMA,
        ],
        compiler_params=pltpu.CompilerParams(use_tc_tiling_on_sc=False),
    )
    def k(nf0_hbm, nf1_hbm, src_hbm, dst_hbm, zd_hbm, acc_out,
          acc_sh, idxs, idxd, idxts, idxtd,
          rowb0, rowb1, rowb2, rowb3, rowb4, rowb5,
          gsem, ssem, isem):
        rowb = [rowb0, rowb1, rowb2, rowb3, rowb4, rowb5]
        c = lax.axis_index("c")
        s = lax.axis_index("s")

        r0 = s * ROWS_PER_TILE
        rt0 = NS * ROWS_PER_TILE
        pltpu.sync_copy(zd_hbm, acc_sh.at[pl.ds(r0, ROWS_PER_TILE)])

        @pl.when(s == 0)
        def _zero_tail():
            pltpu.sync_copy(zd_hbm.at[pl.ds(0, ROW_TAIL)],
                            acc_sh.at[pl.ds(rt0, ROW_TAIL)])

        plsc.subcore_barrier()

        chunk0 = s * CHUNKS_PER_TILE

        pltpu.async_copy(src_hbm.at[pl.ds(chunk0, BLK)], idxs.at[0], isem)
        pltpu.async_copy(dst_hbm.at[pl.ds(chunk0, BLK)], idxd.at[0], isem)

        def block_body(b, carry):
            row0 = chunk0 + b * BLK
            p = lax.rem(b, 2)
            pltpu.make_async_copy(src_hbm.at[pl.ds(row0, BLK)],
                                  idxs.at[p], isem).wait()
            pltpu.make_async_copy(dst_hbm.at[pl.ds(row0, BLK)],
                                  idxd.at[p], isem).wait()

            @pl.when(b + 1 < N_BLKS)
            def _prefetch_idx():
                pltpu.async_copy(src_hbm.at[pl.ds(row0 + BLK, BLK)],
                                 idxs.at[1 - p], isem)
                pltpu.async_copy(dst_hbm.at[pl.ds(row0 + BLK, BLK)],
                                 idxd.at[1 - p], isem)

            sd = {}

            def start_g(i):
                @pl.when(c == 0)
                def _g0(i=i):
                    pltpu.async_copy(
                        nf0_hbm.at[idxs.at[p, i]], rowb[i % NBUF], gsem)

                @pl.when(c == 1)
                def _g1(i=i):
                    pltpu.async_copy(
                        nf1_hbm.at[idxs.at[p, i]], rowb[i % NBUF], gsem)

            def wait_g(i):
                pltpu.make_async_copy(
                    nf0_hbm.at[idxs.at[p, i]], rowb[i % NBUF], gsem).wait()

            for i in range(NBUF):
                start_g(i)

            for i in range(BLK):
                q = i % NBUF
                wait_g(i)
                sd[i] = pltpu.async_copy(
                    rowb[q], acc_sh.at[idxd.at[p, i]], ssem, add=True)
                if i + NBUF < BLK:
                    sd[i].wait()
                    start_g(i + NBUF)

            for i in range(BLK - NBUF, BLK):
                sd[i].wait()

            return carry

        lax.fori_loop(0, N_BLKS, block_body, 0)

        @pl.when(s < TAIL_CHUNKS)
        def _tail():
            row = NS * CHUNKS_PER_TILE + s
            pltpu.sync_copy(src_hbm.at[pl.ds(row, 1)], idxts)
            pltpu.sync_copy(dst_hbm.at[pl.ds(row, 1)], idxtd)

            @pl.when(c == 0)
            def _tg0():
                pltpu.async_copy(nf0_hbm.at[idxts.at[0]], rowb0, gsem)

            @pl.when(c == 1)
            def _tg1():
                pltpu.async_copy(nf1_hbm.at[idxts.at[0]], rowb0, gsem)

            pltpu.make_async_copy(
                nf0_hbm.at[idxts.at[0]], rowb0, gsem).wait()
            pltpu.sync_copy(rowb0, acc_sh.at[idxtd.at[0]], add=True)

        plsc.subcore_barrier()

        pltpu.sync_copy(acc_sh.at[pl.ds(r0, ROWS_PER_TILE)],
                        acc_out.at[c, pl.ds(r0, ROWS_PER_TILE)])

        @pl.when(s == 0)
        def _write_tail():
            pltpu.sync_copy(acc_sh.at[pl.ds(rt0, ROW_TAIL)],
                            acc_out.at[c, pl.ds(rt0, ROW_TAIL)])

    return k(nf0, nf1, src2d, dst2d, zeros_d)


def _sc_edge_accumulate(ef, dst2d, zeros_e, ones_e):
    @functools.partial(
        pl.kernel,
        out_type=[
            jax.ShapeDtypeStruct((NC, N_NODES, D_EDGE), jnp.float32),
            jax.ShapeDtypeStruct((NC, N_NODES, D_EDGE), jnp.float32),
        ],
        mesh=plsc.VectorSubcoreMesh(**_MESH),
        scratch_types=[
            pltpu.VMEM_SHARED((N_NODES, D_EDGE), jnp.float32),
            pltpu.VMEM_SHARED((N_NODES, D_EDGE), jnp.float32),
            pltpu.VMEM((2, BLK, CHUNK), jnp.int32),
            pltpu.VMEM((1, CHUNK), jnp.int32),
            pltpu.VMEM((2, BLK // 2, CHUNK, D_EDGE), jnp.float32),
            pltpu.VMEM((CHUNK, D_EDGE), jnp.float32),
            pltpu.SemaphoreType.DMA,
            pltpu.SemaphoreType.DMA,
            pltpu.SemaphoreType.DMA,
            pltpu.SemaphoreType.DMA,
        ],
        compiler_params=pltpu.CompilerParams(use_tc_tiling_on_sc=False),
    )
    def k(ef_hbm, dst_hbm, ze_hbm, oe_hbm, efacc_out, one_out,
          efacc_sh, one_sh, dstblk, idxtd, efblk, onesbuf,
          efsem, esem, osem, dsem):
        c = lax.axis_index("c")
        s = lax.axis_index("s")

        r0 = s * ROWS_PER_TILE
        rt0 = NS * ROWS_PER_TILE
        pltpu.sync_copy(ze_hbm, efacc_sh.at[pl.ds(r0, ROWS_PER_TILE)])
        pltpu.sync_copy(ze_hbm, one_sh.at[pl.ds(r0, ROWS_PER_TILE)])

        @pl.when(s == 0)
        def _zero_tail():
            pltpu.sync_copy(ze_hbm.at[pl.ds(0, ROW_TAIL)],
                            efacc_sh.at[pl.ds(rt0, ROW_TAIL)])
            pltpu.sync_copy(ze_hbm.at[pl.ds(0, ROW_TAIL)],
                            one_sh.at[pl.ds(rt0, ROW_TAIL)])

        pltpu.sync_copy(oe_hbm, onesbuf)
        plsc.subcore_barrier()

        chunk0 = s * CHUNKS_PER_TILE

        pltpu.async_copy(dst_hbm.at[pl.ds(chunk0, BLK)], dstblk.at[0], dsem)
        for t in range(BLK // 2):
            pltpu.async_copy(
                ef_hbm.at[pl.ds((chunk0 + 2 * t + c) * CHUNK, CHUNK)],
                efblk.at[0, t], efsem)

        def block_body(b, carry):
            row0 = chunk0 + b * BLK
            p = lax.rem(b, 2)

            @pl.when(b + 1 < N_BLKS)
            def _prefetch():
                pltpu.async_copy(dst_hbm.at[pl.ds(row0 + BLK, BLK)],
                                 dstblk.at[1 - p], dsem)
                for t in range(BLK // 2):
                    pltpu.async_copy(
                        ef_hbm.at[pl.ds((row0 + BLK + 2 * t + c) * CHUNK,
                                        CHUNK)],
                        efblk.at[1 - p, t], efsem)

            pltpu.make_async_copy(dst_hbm.at[pl.ds(row0, BLK)],
                                  dstblk.at[p], dsem).wait()

            e2, o2 = {}, {}
            for t in range(BLK // 2):
                pltpu.make_async_copy(
                    ef_hbm.at[pl.ds(row0 * CHUNK, CHUNK)],
                    efblk.at[p, t], efsem).wait()
                e2[t] = pltpu.async_copy(
                    efblk.at[p, t], efacc_sh.at[dstblk.at[p, 2 * t + c]],
                    esem, add=True)
                o2[t] = pltpu.async_copy(
                    onesbuf, one_sh.at[dstblk.at[p, 2 * t + c]], osem,
                    add=True)
            for t in range(BLK // 2):
                e2[t].wait()
                o2[t].wait()
            return carry

        lax.fori_loop(0, N_BLKS, block_body, 0)

        @pl.when(jnp.logical_and(s < TAIL_CHUNKS, c == 0))
        def _tail():
            row = NS * CHUNKS_PER_TILE + s
            off = row * CHUNK
            pltpu.sync_copy(dst_hbm.at[pl.ds(row, 1)], idxtd)
            pltpu.sync_copy(ef_hbm.at[pl.ds(off, CHUNK)], efblk.at[0, 0])
            pltpu.sync_copy(efblk.at[0, 0], efacc_sh.at[idxtd.at[0]], add=True)
            pltpu.sync_copy(onesbuf, one_sh.at[idxtd.at[0]], add=True)

        plsc.subcore_barrier()

        pltpu.sync_copy(efacc_sh.at[pl.ds(r0, ROWS_PER_TILE)],
                        efacc_out.at[c, pl.ds(r0, ROWS_PER_TILE)])
        pltpu.sync_copy(one_sh.at[pl.ds(r0, ROWS_PER_TILE)],
                        one_out.at[c, pl.ds(r0, ROWS_PER_TILE)])

        @pl.when(s == 0)
        def _write_tail():
            pltpu.sync_copy(efacc_sh.at[pl.ds(rt0, ROW_TAIL)],
                            efacc_out.at[c, pl.ds(rt0, ROW_TAIL)])
            pltpu.sync_copy(one_sh.at[pl.ds(rt0, ROW_TAIL)],
                            one_out.at[c, pl.ds(rt0, ROW_TAIL)])

    return k(ef, dst2d, zeros_e, ones_e)


def _tc_combine(acc, efacc, one, W_node, W_edge, b_node, b_edge):
    BR = 1000
    grid = (N_NODES // BR,)

    def body(acc_ref, ef_ref, one_ref, wn_ref, we_ref, bn_ref, be_ref, out_ref):
        wn = wn_ref[...]
        cnt = one_ref[0][:, 0:1] + one_ref[1][:, 0:1]
        out_ref[...] = (
            jnp.dot(acc_ref[0], wn[:DH], preferred_element_type=jnp.float32)
            + jnp.dot(acc_ref[1], wn[DH:], preferred_element_type=jnp.float32)
            + jnp.dot(ef_ref[0] + ef_ref[1], we_ref[...],
                      preferred_element_type=jnp.float32)
            + cnt * (bn_ref[...] + be_ref[...])
        )

    return pl.pallas_call(
        body,
        grid=grid,
        in_specs=[
            pl.BlockSpec((NC, BR, DH), lambda i: (0, i, 0)),
            pl.BlockSpec((NC, BR, D_EDGE), lambda i: (0, i, 0)),
            pl.BlockSpec((NC, BR, D_EDGE), lambda i: (0, i, 0)),
            pl.BlockSpec((D_NODE, D_OUT), lambda i: (0, 0)),
            pl.BlockSpec((D_EDGE, D_OUT), lambda i: (0, 0)),
            pl.BlockSpec((1, D_OUT), lambda i: (0, 0)),
            pl.BlockSpec((1, D_OUT), lambda i: (0, 0)),
        ],
        out_specs=pl.BlockSpec((BR, D_OUT), lambda i: (i, 0)),
        out_shape=jax.ShapeDtypeStruct((N_NODES, D_OUT), jnp.float32),
    )(acc, efacc, one, W_node, W_edge,
      b_node.reshape(1, D_OUT), b_edge.reshape(1, D_OUT))


def kernel(node_feats, edge_index, edge_feats, W_node, b_node, W_edge, b_edge):
    ei = edge_index.astype(jnp.int32)
    src2d = ei[0].reshape(N_CHUNKS, CHUNK)
    dst2d = ei[1].reshape(N_CHUNKS, CHUNK)
    nf0 = node_feats[:, :DH]
    nf1 = node_feats[:, DH:]
    zeros_d = jnp.zeros((ROWS_PER_TILE, DH), jnp.float32)
    zeros_e = jnp.zeros((ROWS_PER_TILE, D_EDGE), jnp.float32)
    ones_e = jnp.ones((CHUNK, D_EDGE), jnp.float32)
    acc = _sc_node_accumulate(nf0, nf1, src2d, dst2d, zeros_d)
    efacc, one = _sc_edge_accumulate(edge_feats, dst2d, zeros_e, ones_e)
    return _tc_combine(acc, efacc, one, W_node, W_edge, b_node, b_edge)
